# Initial kernel scaffold; baseline (speedup 1.0000x reference)
import jax
import jax.numpy as jnp
from jax.experimental import pallas as pl


def kernel(inputs, Win, Wres, Wout):
    raise NotImplementedError("write your pallas kernel here")


# speedup vs baseline: 3.2854x; 3.2854x over previous
import jax
import jax.numpy as jnp
from jax.experimental import pallas as pl
from jax.experimental.pallas import tpu as pltpu

B, T, D, N = 32, 256, 128, 2000
NP = 2048


def _scan_body(x_ref, win_ref, wres_ref, wout_ref, out_ref, h_ref):
    t = pl.program_id(0)

    @pl.when(t == 0)
    def _():
        h_ref[...] = jnp.zeros_like(h_ref)

    x_t = x_ref[0]
    u = jnp.dot(x_t, win_ref[...], preferred_element_type=jnp.float32)
    h = h_ref[...]
    pre = u + jnp.dot(h, wres_ref[...], preferred_element_type=jnp.float32)
    hn = jnp.tanh(pre)
    h_ref[...] = hn
    out_ref[0] = jnp.dot(hn, wout_ref[...], preferred_element_type=jnp.float32)


def kernel(inputs, Win, Wres, Wout):
    x = jnp.transpose(inputs, (1, 0, 2))
    win = jnp.zeros((D, NP), jnp.float32).at[:, :N].set(Win)
    wres = jnp.zeros((NP, NP), jnp.float32).at[:N, :N].set(Wres)
    wout = jnp.zeros((NP, D), jnp.float32).at[:N, :].set(Wout)

    out = pl.pallas_call(
        _scan_body,
        grid=(T,),
        in_specs=[
            pl.BlockSpec((1, B, D), lambda t: (t, 0, 0)),
            pl.BlockSpec((D, NP), lambda t: (0, 0)),
            pl.BlockSpec((NP, NP), lambda t: (0, 0)),
            pl.BlockSpec((NP, D), lambda t: (0, 0)),
        ],
        out_specs=pl.BlockSpec((1, B, D), lambda t: (t, 0, 0)),
        out_shape=jax.ShapeDtypeStruct((T, B, D), jnp.float32),
        scratch_shapes=[pltpu.VMEM((B, NP), jnp.float32)],
    )(x, win, wres, wout)
    return jnp.transpose(out, (1, 0, 2))
